# Initial kernel scaffold; baseline (speedup 1.0000x reference)
import functools

import jax
import jax.numpy as jnp
from jax import lax
from jax.experimental import pallas as pl
from jax.experimental.pallas import tpu as pltpu
from jax.experimental.pallas import tpu_sc as plsc

VOCAB = 100000
EMBED = 64
BATCH = 4096
HIST = 50

NC = 2
NS = 16
LANES = 16
NW = NC * NS
B_W = BATCH // NW
NBUF = 4


def _body(idx_hbm, table_hbm, out_hbm, idx_v, rows_v, out_v, *sems):
    wid = lax.axis_index("s") * NC + lax.axis_index("c")
    base = wid * B_W

    pltpu.sync_copy(idx_hbm.at[pl.ds(base, B_W)], idx_v)

    def fire(s, b):
        return pltpu.async_copy(table_hbm.at[idx_v.at[s]], rows_v.at[b], sems[b])

    for b in range(NBUF):
        fire(b, b)

    n_groups = B_W // NBUF

    def group(gi, carry):
        for b in range(NBUF):
            s = gi * NBUF + b
            pltpu.make_async_copy(table_hbm.at[idx_v.at[s]], rows_v.at[b],
                                  sems[b]).wait()
            def tok(t, accs):
                return tuple(
                    a + rows_v[b, t, pl.ds(j * LANES, LANES)]
                    for j, a in enumerate(accs)
                )
            init = tuple(rows_v[b, 0, pl.ds(j * LANES, LANES)]
                         for j in range(EMBED // LANES))
            accs = lax.fori_loop(1, HIST, tok, init)
            for j in range(EMBED // LANES):
                out_v[s, pl.ds(j * LANES, LANES)] = accs[j] * (1.0 / HIST)
            @pl.when(s + NBUF < B_W)
            def _():
                fire(s + NBUF, b)
        return carry

    lax.fori_loop(0, n_groups, group, 0)

    pltpu.sync_copy(out_v, out_hbm.at[pl.ds(base, B_W)])


@jax.jit
def _emb_mean(idx, table):
    mesh = plsc.VectorSubcoreMesh(core_axis_name="c", subcore_axis_name="s")
    return pl.kernel(
        _body,
        out_type=jax.ShapeDtypeStruct((BATCH, EMBED), jnp.float32),
        mesh=mesh,
        scratch_types=[
            pltpu.VMEM((B_W, HIST), jnp.int32),
            pltpu.VMEM((NBUF, HIST, EMBED), jnp.float32),
            pltpu.VMEM((B_W, EMBED), jnp.float32),
        ] + [pltpu.SemaphoreType.DMA] * NBUF,
    )(idx, table)


def kernel(input_var, table):
    return _emb_mean(input_var.astype(jnp.int32), table)


# speedup vs baseline: 9.1636x; 9.1636x over previous
import functools

import jax
import jax.numpy as jnp
from jax import lax
from jax.experimental import pallas as pl
from jax.experimental.pallas import tpu as pltpu
from jax.experimental.pallas import tpu_sc as plsc

VOCAB = 100000
EMBED = 64
BATCH = 4096
HIST = 50

NC = 2
NS = 16
LANES = 16
NW = NC * NS
B_W = BATCH // NW
NBUF = 4


def _body(idx_hbm, table_hbm, out_hbm, idx_v, rows_v, out_v, *sems):
    wid = lax.axis_index("s") * NC + lax.axis_index("c")
    base = wid * B_W

    pltpu.sync_copy(idx_hbm.at[pl.ds(base, B_W)], idx_v)

    def fire(s, b):
        return pltpu.async_copy(table_hbm.at[idx_v.at[s]], rows_v.at[b], sems[b])

    for b in range(NBUF):
        fire(b, b)

    n_groups = B_W // NBUF

    def group(gi, carry):
        for b in range(NBUF):
            s = gi * NBUF + b
            pltpu.make_async_copy(table_hbm.at[idx_v.at[s]], rows_v.at[b],
                                  sems[b]).wait()
            def tok(t, accs):
                return tuple(
                    a + rows_v[b, t, pl.ds(j * LANES, LANES)]
                    for j, a in enumerate(accs)
                )
            init = tuple(rows_v[b, 0, pl.ds(j * LANES, LANES)]
                         for j in range(EMBED // LANES))
            accs = lax.fori_loop(1, HIST, tok, init)
            for j in range(EMBED // LANES):
                out_v[s, pl.ds(j * LANES, LANES)] = accs[j] * (1.0 / HIST)
            @pl.when(s + NBUF < B_W)
            def _():
                fire(s + NBUF, b)
        return carry

    lax.fori_loop(0, n_groups, group, 0)

    pltpu.sync_copy(out_v, out_hbm.at[pl.ds(base, B_W)])


@jax.jit
def _emb_mean(idx, table):
    mesh = plsc.VectorSubcoreMesh(core_axis_name="c", subcore_axis_name="s")
    return pl.kernel(
        _body,
        out_type=jax.ShapeDtypeStruct((BATCH, EMBED), jnp.float32),
        mesh=mesh,
        compiler_params=pltpu.CompilerParams(use_tc_tiling_on_sc=False),
        scratch_types=[
            pltpu.VMEM((B_W, HIST), jnp.int32),
            pltpu.VMEM((NBUF, HIST, EMBED), jnp.float32),
            pltpu.VMEM((B_W, EMBED), jnp.float32),
        ] + [pltpu.SemaphoreType.DMA] * NBUF,
    )(idx, table)


def kernel(input_var, table):
    return _emb_mean(input_var.astype(jnp.int32), table)


# speedup vs baseline: 9.9722x; 1.0882x over previous
import jax
import jax.numpy as jnp
from jax import lax
from jax.experimental import pallas as pl
from jax.experimental.pallas import tpu as pltpu
from jax.experimental.pallas import tpu_sc as plsc

VOCAB = 100000
EMBED = 64
BATCH = 4096
HIST = 50

NC = 2
NS = 16
LANES = 16
NJ = EMBED // LANES
NW = NC * NS
B_W = BATCH // NW
SPU = 2
IPU = SPU * HIST
U_W = B_W // SPU
NBUF = 4


def _body(idx_hbm, table_hbm, out_hbm, idx_v, rows_v, out_v, *sems):
    wid = lax.axis_index("s") * NC + lax.axis_index("c")
    ubase = wid * U_W

    pltpu.sync_copy(idx_hbm.at[pl.ds(ubase, U_W)], idx_v)

    def fire(u, b):
        return pltpu.async_copy(table_hbm.at[idx_v.at[u]], rows_v.at[b], sems[b])

    for b in range(NBUF):
        fire(b, b)

    def group(gi, carry):
        for b in range(NBUF):
            u = gi * NBUF + b
            pltpu.make_async_copy(table_hbm.at[idx_v.at[u]], rows_v.at[b],
                                  sems[b]).wait()
            for p in range(SPU):
                base_t = p * HIST

                def tok(i, accs):
                    t = 2 * i
                    return tuple(
                        accs[k * NJ + j]
                        + rows_v[b, base_t + t + k, pl.ds(j * LANES, LANES)]
                        for k in range(2) for j in range(NJ)
                    )

                zero = jnp.zeros((LANES,), jnp.float32)
                accs = lax.fori_loop(0, HIST // 2, tok, (zero,) * (2 * NJ))
                s = SPU * u + p
                for j in range(NJ):
                    out_v[s, pl.ds(j * LANES, LANES)] = (
                        (accs[j] + accs[NJ + j]) * (1.0 / HIST))
            @pl.when(u + NBUF < U_W)
            def _():
                fire(u + NBUF, b)
        return carry

    lax.fori_loop(0, U_W // NBUF, group, 0)

    pltpu.sync_copy(out_v, out_hbm.at[pl.ds(wid * B_W, B_W)])


@jax.jit
def _emb_mean(idx, table):
    mesh = plsc.VectorSubcoreMesh(core_axis_name="c", subcore_axis_name="s")
    return pl.kernel(
        _body,
        out_type=jax.ShapeDtypeStruct((BATCH, EMBED), jnp.float32),
        mesh=mesh,
        compiler_params=pltpu.CompilerParams(use_tc_tiling_on_sc=False),
        scratch_types=[
            pltpu.VMEM((U_W, IPU), jnp.int32),
            pltpu.VMEM((NBUF, IPU, EMBED), jnp.float32),
            pltpu.VMEM((B_W, EMBED), jnp.float32),
        ] + [pltpu.SemaphoreType.DMA] * NBUF,
    )(idx, table)


def kernel(input_var, table):
    idx = input_var.astype(jnp.int32).reshape(BATCH // SPU, IPU)
    return _emb_mean(idx, table)
